# baseline (speedup 1.0000x reference)
import functools

import jax
import jax.numpy as jnp
from jax import lax
from jax.experimental import pallas as pl
from jax.experimental.pallas import tpu as pltpu, tpu_sc as plsc

_PL = 4
_PPW = 56


def _sc_relu(x_hbm, o_hbm, buf, sem_in, sem_out):
    c = lax.axis_index("c")
    s = lax.axis_index("s")
    wid = s * 2 + c
    i = wid // 8
    j = (wid // 4) % 2
    k0 = (wid % 4) * _PPW

    def chunk(t, _):
        src = x_hbm.at[i, j, pl.ds(k0 + t * _PL, _PL)]
        dst = o_hbm.at[i, j, pl.ds(k0 + t * _PL, _PL)]
        pltpu.async_copy(src, buf, sem_in).wait()

        def row(r, _):
            p = r // 96
            q = r % 96
            for l in range(14):
                v = buf[p, q, pl.ds(l * 16, 16)]
                buf[p, q, pl.ds(l * 16, 16)] = jnp.maximum(v, 0.0)
            return 0

        lax.fori_loop(0, _PL * 96, row, 0)
        pltpu.async_copy(buf, dst, sem_out).wait()
        return 0

    lax.fori_loop(0, _PPW // _PL, chunk, 0)


def kernel(inputs):
    b0, b1, h, w, c = inputs.shape
    xt = inputs.transpose(0, 1, 2, 4, 3)
    mesh = plsc.VectorSubcoreMesh(core_axis_name="c", subcore_axis_name="s")
    k = functools.partial(
        pl.kernel,
        mesh=mesh,
        out_type=jax.ShapeDtypeStruct(xt.shape, jnp.float32),
        scratch_types=[
            pltpu.VMEM((_PL, c, w), jnp.float32),
            pltpu.SemaphoreType.DMA,
            pltpu.SemaphoreType.DMA,
        ],
    )(_sc_relu)
    out = k(xt)
    return out.transpose(0, 1, 2, 4, 3)


# speedup vs baseline: 1.3177x; 1.3177x over previous
import functools

import jax
import jax.numpy as jnp
from jax import lax
from jax.experimental import pallas as pl
from jax.experimental.pallas import tpu as pltpu, tpu_sc as plsc

_PPW = 56
_C = 96
_W = 224


def _sc_relu(x_hbm, o_hbm, ibufs, obufs, isems, osems):
    c = lax.axis_index("c")
    s = lax.axis_index("s")
    wid = s * 2 + c
    i = wid // 8
    j = (wid // 4) % 2
    k0 = (wid % 4) * _PPW

    def in_copy(step, b):
        return pltpu.make_async_copy(
            x_hbm.at[i, j, k0 + step], ibufs[b], isems[b]
        )

    def out_copy(step, b):
        return pltpu.make_async_copy(
            obufs[b], o_hbm.at[i, j, k0 + step], osems[b]
        )

    in_copy(0, 0).start()
    in_copy(1, 1).start()

    def round_(r, _):
        for b in range(2):
            step = r * 2 + b

            @pl.when(r > 0)
            def _():
                out_copy(step - 2, b).wait()

            in_copy(step, b).wait()

            def row(q, _):
                for l in range(14):
                    obufs[b][q, pl.ds(l * 16, 16)] = jnp.maximum(
                        ibufs[b][q, pl.ds(l * 16, 16)], 0.0
                    )
                return 0

            lax.fori_loop(0, _C, row, 0, unroll=2)
            out_copy(step, b).start()

            @pl.when(step + 2 < _PPW)
            def _():
                in_copy(step + 2, b).start()

        return 0

    lax.fori_loop(0, _PPW // 2, round_, 0)
    out_copy(_PPW - 2, 0).wait()
    out_copy(_PPW - 1, 1).wait()


def kernel(inputs):
    b0, b1, h, w, c = inputs.shape
    xt = inputs.transpose(0, 1, 2, 4, 3)
    mesh = plsc.VectorSubcoreMesh(core_axis_name="c", subcore_axis_name="s")
    k = functools.partial(
        pl.kernel,
        mesh=mesh,
        out_type=jax.ShapeDtypeStruct(xt.shape, jnp.float32),
        scratch_types=[
            [pltpu.VMEM((c, w), jnp.float32) for _ in range(2)],
            [pltpu.VMEM((c, w), jnp.float32) for _ in range(2)],
            [pltpu.SemaphoreType.DMA for _ in range(2)],
            [pltpu.SemaphoreType.DMA for _ in range(2)],
        ],
    )(_sc_relu)
    out = k(xt)
    return out.transpose(0, 1, 2, 4, 3)
